# baseline (speedup 1.0000x reference)
import functools

import jax
import jax.numpy as jnp
from jax import lax
from jax.experimental import pallas as pl
from jax.experimental.pallas import tpu as pltpu
from jax.experimental.pallas import tpu_sc as plsc

N_MESH = 10000
N_GRID = 10000
E = 160000
D = 256
H = 512

NC = 2
NS = 16
NW = NC * NS
CHUNK = 128
NCHUNKS = E // CHUNK

NPAD = 10240
NHALF = NPAD // NC
FCOL = D // NS


def _gather_body(mesh_hbm, grid_hbm, src_hbm, dst_hbm, gm_hbm, gg_hbm,
                 idx_v, rows_v, sem):
    c = lax.axis_index("c")
    s = lax.axis_index("s")
    wid = s * NC + c
    niters = (NCHUNKS + NW - 1) // NW

    def body(i, carry):
        chunk = wid + i * NW

        @pl.when(chunk < NCHUNKS)
        def _():
            base = chunk * CHUNK
            pltpu.sync_copy(src_hbm.at[pl.ds(base, CHUNK)], idx_v)
            pltpu.async_copy(mesh_hbm.at[idx_v], rows_v, sem).wait()
            pltpu.sync_copy(rows_v, gm_hbm.at[pl.ds(base, CHUNK)])
            pltpu.sync_copy(dst_hbm.at[pl.ds(base, CHUNK)], idx_v)
            pltpu.async_copy(grid_hbm.at[idx_v], rows_v, sem).wait()
            pltpu.sync_copy(rows_v, gg_hbm.at[pl.ds(base, CHUNK)])

        return carry

    lax.fori_loop(0, niters, body, 0)


@functools.cache
def _make_gather():
    mesh = plsc.VectorSubcoreMesh(
        core_axis_name="c", subcore_axis_name="s",
        num_cores=NC, num_subcores=NS)
    return pl.kernel(
        _gather_body,
        out_type=[jax.ShapeDtypeStruct((E, D), jnp.float32),
                  jax.ShapeDtypeStruct((E, D), jnp.float32)],
        mesh=mesh,
        scratch_types=[pltpu.VMEM((CHUNK,), jnp.int32),
                       pltpu.VMEM((CHUNK, D), jnp.float32),
                       pltpu.SemaphoreType.DMA],
    )


def _scatter_body(rows_hbm, dst_hbm, zeros_hbm, agg_hbm,
                  rows_v, didx_v, acc):
    c = lax.axis_index("c")
    s = lax.axis_index("s")
    lo = c * NHALF
    col0 = s * FCOL

    pltpu.sync_copy(zeros_hbm, acc)

    def body(i, carry):
        base = i * CHUNK
        pltpu.sync_copy(dst_hbm.at[pl.ds(base, CHUNK)], didx_v)
        pltpu.sync_copy(rows_hbm.at[pl.ds(col0, FCOL), pl.ds(base, CHUNK)],
                        rows_v)
        for v in range(CHUNK // 16):
            d16 = didx_v[pl.ds(v * 16, 16)]
            loc = d16 - lo
            m = (loc >= 0) & (loc < NHALF)
            loc = jnp.where(m, loc, 0)
            for j in range(FCOL):
                vals = rows_v[j, pl.ds(v * 16, 16)]
                plsc.addupdate_scatter(
                    acc, [jnp.full((16,), j, jnp.int32), loc], vals, mask=m)
        return carry

    lax.fori_loop(0, NCHUNKS, body, 0)

    pltpu.sync_copy(acc, agg_hbm.at[pl.ds(col0, FCOL), pl.ds(lo, NHALF)])


@functools.cache
def _make_scatter():
    mesh = plsc.VectorSubcoreMesh(
        core_axis_name="c", subcore_axis_name="s",
        num_cores=NC, num_subcores=NS)
    return pl.kernel(
        _scatter_body,
        out_type=[jax.ShapeDtypeStruct((D, NPAD), jnp.float32)],
        mesh=mesh,
        compiler_params=pltpu.CompilerParams(needs_layout_passes=False),
        scratch_types=[pltpu.VMEM((FCOL, CHUNK), jnp.float32),
                       pltpu.VMEM((CHUNK,), jnp.int32),
                       pltpu.VMEM((FCOL, NHALF), jnp.float32)],
    )


BE = 1280
BN = 1024

_F32 = jnp.float32


def _edge_body(ef_ref, gm_ref, gg_ref, w1_ref, b1_ref, w2_ref, b2_ref,
               g_ref, bt_ref, out_ref):
    dn = (((0,), (1,)), ((), ()))
    x = lax.dot_general(w1_ref[0:D, :], ef_ref[...], dn,
                        preferred_element_type=_F32)
    x += lax.dot_general(w1_ref[D:2 * D, :], gm_ref[...], dn,
                         preferred_element_type=_F32)
    x += lax.dot_general(w1_ref[2 * D:3 * D, :], gg_ref[...], dn,
                         preferred_element_type=_F32)
    x += b1_ref[...]
    h = jax.nn.silu(x)
    y = lax.dot_general(w2_ref[...], h, (((0,), (0,)), ((), ())),
                        preferred_element_type=_F32) + b2_ref[...]
    mu = jnp.mean(y, axis=0, keepdims=True)
    var = jnp.mean((y - mu) ** 2, axis=0, keepdims=True)
    out_ref[...] = (y - mu) * lax.rsqrt(var + 1e-5) * g_ref[...] + bt_ref[...]


def _node_body(aggt_ref, gn_ref, w1_ref, b1_ref, w2_ref, b2_ref,
               g_ref, bt_ref, out_ref):
    x = lax.dot_general(aggt_ref[...], w1_ref[0:D, :], (((0,), (0,)), ((), ())),
                        preferred_element_type=_F32)
    x += jnp.dot(gn_ref[...], w1_ref[D:2 * D, :], preferred_element_type=_F32)
    x += b1_ref[...]
    h = jax.nn.silu(x)
    y = jnp.dot(h, w2_ref[...], preferred_element_type=_F32) + b2_ref[...]
    mu = jnp.mean(y, axis=-1, keepdims=True)
    var = jnp.mean((y - mu) ** 2, axis=-1, keepdims=True)
    out_ref[...] = ((y - mu) * lax.rsqrt(var + 1e-5) * g_ref[...] + bt_ref[...]
                    + gn_ref[...])


def _row_spec(r, c_):
    return pl.BlockSpec((r, c_), lambda i: (i, 0))


def _col_spec(r, c_):
    return pl.BlockSpec((r, c_), lambda i: (0, i))


def _full_spec(r, c_):
    return pl.BlockSpec((r, c_), lambda i: (0, 0))


_edge_mlp = pl.pallas_call(
    _edge_body,
    grid=(E // BE,),
    in_specs=[_row_spec(BE, D), _row_spec(BE, D), _row_spec(BE, D),
              _full_spec(3 * D, H), _full_spec(H, 1),
              _full_spec(H, D), _full_spec(D, 1),
              _full_spec(D, 1), _full_spec(D, 1)],
    out_specs=_col_spec(D, BE),
    out_shape=jax.ShapeDtypeStruct((D, E), jnp.float32),
)

_node_mlp = pl.pallas_call(
    _node_body,
    grid=(NPAD // BN,),
    in_specs=[_col_spec(D, BN), _row_spec(BN, D),
              _full_spec(2 * D, H), _full_spec(1, H),
              _full_spec(H, D), _full_spec(1, D),
              _full_spec(1, D), _full_spec(1, D)],
    out_specs=_row_spec(BN, D),
    out_shape=jax.ShapeDtypeStruct((NPAD, D), jnp.float32),
)


def kernel(m2g_efeat, grid_nfeat, mesh_nfeat, edge_index,
           eW1, eb1, eW2, eb2, eg, ebt,
           nW1, nb1, nW2, nb2, ng, nbt):
    src = edge_index[0]
    dst = edge_index[1]

    gm, gg = _make_gather()(mesh_nfeat, grid_nfeat, src, dst)

    e_out_t = _edge_mlp(m2g_efeat, gm, gg, eW1,
                        eb1.reshape(H, 1), eW2, eb2.reshape(D, 1),
                        eg.reshape(D, 1), ebt.reshape(D, 1))

    zeros = jnp.zeros((FCOL, NHALF), dtype=jnp.float32)
    (agg_t,) = _make_scatter()(e_out_t, dst, zeros)

    gn_pad = jnp.pad(grid_nfeat, ((0, NPAD - N_GRID), (0, 0)))
    out = _node_mlp(agg_t, gn_pad, nW1,
                    nb1.reshape(1, H), nW2, nb2.reshape(1, D),
                    ng.reshape(1, D), nbt.reshape(1, D))
    return out[:N_GRID]


# speedup vs baseline: 1.8795x; 1.8795x over previous
import functools

import jax
import jax.numpy as jnp
from jax import lax
from jax.experimental import pallas as pl
from jax.experimental.pallas import tpu as pltpu
from jax.experimental.pallas import tpu_sc as plsc

N_MESH = 10000
N_GRID = 10000
E = 160000
D = 256
H = 512

NC = 2
NS = 16
NW = NC * NS
CHUNK = 128
NCHUNKS = E // CHUNK
NPAIRS = NCHUNKS // 2

NPAD = 10240
NHALF = NPAD // NC
FCOL = D // NS


def _gather_body(mesh_hbm, grid_hbm, src_hbm, dst_hbm, gm_hbm, gg_hbm,
                 idx_v, rows_v, sem):
    c = lax.axis_index("c")
    s = lax.axis_index("s")
    wid = s * NC + c
    niters = (NCHUNKS + NW - 1) // NW

    def body(i, carry):
        chunk = wid + i * NW

        @pl.when(chunk < NCHUNKS)
        def _():
            base = chunk * CHUNK
            pltpu.sync_copy(src_hbm.at[pl.ds(base, CHUNK)], idx_v)
            pltpu.async_copy(mesh_hbm.at[idx_v], rows_v, sem).wait()
            pltpu.sync_copy(rows_v, gm_hbm.at[pl.ds(base, CHUNK)])
            pltpu.sync_copy(dst_hbm.at[pl.ds(base, CHUNK)], idx_v)
            pltpu.async_copy(grid_hbm.at[idx_v], rows_v, sem).wait()
            pltpu.sync_copy(rows_v, gg_hbm.at[pl.ds(base, CHUNK)])

        return carry

    lax.fori_loop(0, niters, body, 0)


@functools.cache
def _make_gather():
    mesh = plsc.VectorSubcoreMesh(
        core_axis_name="c", subcore_axis_name="s",
        num_cores=NC, num_subcores=NS)
    return pl.kernel(
        _gather_body,
        out_type=[jax.ShapeDtypeStruct((E, D), jnp.float32),
                  jax.ShapeDtypeStruct((E, D), jnp.float32)],
        mesh=mesh,
        scratch_types=[pltpu.VMEM((CHUNK,), jnp.int32),
                       pltpu.VMEM((CHUNK, D), jnp.float32),
                       pltpu.SemaphoreType.DMA],
    )


def _scatter_chunk(didx_v, rows_v, acc, lo):
    for v in range(CHUNK // 16):
        d16 = didx_v[pl.ds(v * 16, 16)]
        loc = d16 - lo
        m = (loc >= 0) & (loc < NHALF)
        loc = jnp.where(m, loc, 0)
        for j in range(FCOL):
            vals = rows_v[j, pl.ds(v * 16, 16)]
            plsc.addupdate_scatter(
                acc, [jnp.full((16,), j, jnp.int32), loc], vals, mask=m)


def _scatter_body(rows_hbm, dst_hbm, zeros_hbm, agg_hbm,
                  rows_v0, rows_v1, didx_v0, didx_v1, acc,
                  sem_r0, sem_r1, sem_i0, sem_i1):
    c = lax.axis_index("c")
    s = lax.axis_index("s")
    lo = c * NHALF
    col0 = s * FCOL

    pltpu.sync_copy(zeros_hbm, acc)

    def load(base, didx_v, rows_v, sem_i, sem_r):
        cp_i = pltpu.async_copy(dst_hbm.at[pl.ds(base, CHUNK)], didx_v, sem_i)
        cp_r = pltpu.async_copy(
            rows_hbm.at[pl.ds(col0, FCOL), pl.ds(base, CHUNK)], rows_v, sem_r)
        return cp_i, cp_r

    def wait(base, didx_v, rows_v, sem_i, sem_r):
        pltpu.make_async_copy(dst_hbm.at[pl.ds(base, CHUNK)], didx_v,
                              sem_i).wait()
        pltpu.make_async_copy(
            rows_hbm.at[pl.ds(col0, FCOL), pl.ds(base, CHUNK)], rows_v,
            sem_r).wait()

    load(0, didx_v0, rows_v0, sem_i0, sem_r0)

    def body(p, carry):
        base0 = (2 * p) * CHUNK
        base1 = (2 * p + 1) * CHUNK

        load(base1, didx_v1, rows_v1, sem_i1, sem_r1)
        wait(base0, didx_v0, rows_v0, sem_i0, sem_r0)
        _scatter_chunk(didx_v0, rows_v0, acc, lo)

        @pl.when(p + 1 < NPAIRS)
        def _():
            load(base0 + 2 * CHUNK, didx_v0, rows_v0, sem_i0, sem_r0)

        wait(base1, didx_v1, rows_v1, sem_i1, sem_r1)
        _scatter_chunk(didx_v1, rows_v1, acc, lo)
        return carry

    lax.fori_loop(0, NPAIRS, body, 0)

    pltpu.sync_copy(acc, agg_hbm.at[pl.ds(col0, FCOL), pl.ds(lo, NHALF)])


@functools.cache
def _make_scatter():
    mesh = plsc.VectorSubcoreMesh(
        core_axis_name="c", subcore_axis_name="s",
        num_cores=NC, num_subcores=NS)
    return pl.kernel(
        _scatter_body,
        out_type=[jax.ShapeDtypeStruct((D, NPAD), jnp.float32)],
        mesh=mesh,
        compiler_params=pltpu.CompilerParams(needs_layout_passes=False),
        scratch_types=[pltpu.VMEM((FCOL, CHUNK), jnp.float32),
                       pltpu.VMEM((FCOL, CHUNK), jnp.float32),
                       pltpu.VMEM((CHUNK,), jnp.int32),
                       pltpu.VMEM((CHUNK,), jnp.int32),
                       pltpu.VMEM((FCOL, NHALF), jnp.float32),
                       pltpu.SemaphoreType.DMA,
                       pltpu.SemaphoreType.DMA,
                       pltpu.SemaphoreType.DMA,
                       pltpu.SemaphoreType.DMA],
    )


BE = 1280
BN = 1024

_F32 = jnp.float32


def _edge_body(ef_ref, gm_ref, gg_ref, w1_ref, b1_ref, w2_ref, b2_ref,
               g_ref, bt_ref, out_ref):
    dn = (((0,), (1,)), ((), ()))
    x = lax.dot_general(w1_ref[0:D, :], ef_ref[...], dn,
                        preferred_element_type=_F32)
    x += lax.dot_general(w1_ref[D:2 * D, :], gm_ref[...], dn,
                         preferred_element_type=_F32)
    x += lax.dot_general(w1_ref[2 * D:3 * D, :], gg_ref[...], dn,
                         preferred_element_type=_F32)
    x += b1_ref[...]
    h = jax.nn.silu(x)
    y = lax.dot_general(w2_ref[...], h, (((0,), (0,)), ((), ())),
                        preferred_element_type=_F32) + b2_ref[...]
    mu = jnp.mean(y, axis=0, keepdims=True)
    var = jnp.mean((y - mu) ** 2, axis=0, keepdims=True)
    out_ref[...] = (y - mu) * lax.rsqrt(var + 1e-5) * g_ref[...] + bt_ref[...]


def _node_body(aggt_ref, gn_ref, w1_ref, b1_ref, w2_ref, b2_ref,
               g_ref, bt_ref, out_ref):
    x = lax.dot_general(aggt_ref[...], w1_ref[0:D, :], (((0,), (0,)), ((), ())),
                        preferred_element_type=_F32)
    x += jnp.dot(gn_ref[...], w1_ref[D:2 * D, :], preferred_element_type=_F32)
    x += b1_ref[...]
    h = jax.nn.silu(x)
    y = jnp.dot(h, w2_ref[...], preferred_element_type=_F32) + b2_ref[...]
    mu = jnp.mean(y, axis=-1, keepdims=True)
    var = jnp.mean((y - mu) ** 2, axis=-1, keepdims=True)
    out_ref[...] = ((y - mu) * lax.rsqrt(var + 1e-5) * g_ref[...] + bt_ref[...]
                    + gn_ref[...])


def _row_spec(r, c_):
    return pl.BlockSpec((r, c_), lambda i: (i, 0))


def _col_spec(r, c_):
    return pl.BlockSpec((r, c_), lambda i: (0, i))


def _full_spec(r, c_):
    return pl.BlockSpec((r, c_), lambda i: (0, 0))


_edge_mlp = pl.pallas_call(
    _edge_body,
    grid=(E // BE,),
    in_specs=[_row_spec(BE, D), _row_spec(BE, D), _row_spec(BE, D),
              _full_spec(3 * D, H), _full_spec(H, 1),
              _full_spec(H, D), _full_spec(D, 1),
              _full_spec(D, 1), _full_spec(D, 1)],
    out_specs=_col_spec(D, BE),
    out_shape=jax.ShapeDtypeStruct((D, E), jnp.float32),
)

_node_mlp = pl.pallas_call(
    _node_body,
    grid=(NPAD // BN,),
    in_specs=[_col_spec(D, BN), _row_spec(BN, D),
              _full_spec(2 * D, H), _full_spec(1, H),
              _full_spec(H, D), _full_spec(1, D),
              _full_spec(1, D), _full_spec(1, D)],
    out_specs=_row_spec(BN, D),
    out_shape=jax.ShapeDtypeStruct((NPAD, D), jnp.float32),
)


def kernel(m2g_efeat, grid_nfeat, mesh_nfeat, edge_index,
           eW1, eb1, eW2, eb2, eg, ebt,
           nW1, nb1, nW2, nb2, ng, nbt):
    src = edge_index[0]
    dst = edge_index[1]

    gm, gg = _make_gather()(mesh_nfeat, grid_nfeat, src, dst)

    e_out_t = _edge_mlp(m2g_efeat, gm, gg, eW1,
                        eb1.reshape(H, 1), eW2, eb2.reshape(D, 1),
                        eg.reshape(D, 1), ebt.reshape(D, 1))

    zeros = jnp.zeros((FCOL, NHALF), dtype=jnp.float32)
    (agg_t,) = _make_scatter()(e_out_t, dst, zeros)

    gn_pad = jnp.pad(grid_nfeat, ((0, NPAD - N_GRID), (0, 0)))
    out = _node_mlp(agg_t, gn_pad, nW1,
                    nb1.reshape(1, H), nW2, nb2.reshape(1, D),
                    ng.reshape(1, D), nbt.reshape(1, D))
    return out[:N_GRID]


# speedup vs baseline: 1.9857x; 1.0565x over previous
import functools

import jax
import jax.numpy as jnp
from jax import lax
from jax.experimental import pallas as pl
from jax.experimental.pallas import tpu as pltpu
from jax.experimental.pallas import tpu_sc as plsc

N_MESH = 10000
N_GRID = 10000
E = 160000
D = 256
H = 512

NC = 2
NS = 16
NW = NC * NS
CHUNK = 128
NCHUNKS = E // CHUNK
NPAIRS = NCHUNKS // 2

NPAD = 10240
NHALF = NPAD // NC
FCOL = D // NS


def _gather_body(mesh_hbm, grid_hbm, src_hbm, dst_hbm, gm_hbm, gg_hbm,
                 sidx0, didx0, sidx1, didx1, rows_m, rows_g,
                 sem_si0, sem_di0, sem_si1, sem_di1,
                 sem_gm, sem_gg, sem_sm, sem_sg):
    c = lax.axis_index("c")
    s = lax.axis_index("s")
    wid = s * NC + c
    niters = (NCHUNKS + NW - 1) // NW
    npairs = (niters + 1) // 2

    def idx_load(i, sidx, didx, sem_s, sem_d):
        chunk = wid + i * NW

        @pl.when(chunk < NCHUNKS)
        def _():
            base = chunk * CHUNK
            pltpu.async_copy(src_hbm.at[pl.ds(base, CHUNK)], sidx, sem_s)
            pltpu.async_copy(dst_hbm.at[pl.ds(base, CHUNK)], didx, sem_d)

    def process(i, sidx, didx, sem_s, sem_d):
        chunk = wid + i * NW

        @pl.when(chunk < NCHUNKS)
        def _():
            base = chunk * CHUNK
            pltpu.make_async_copy(src_hbm.at[pl.ds(base, CHUNK)], sidx,
                                  sem_s).wait()
            pltpu.make_async_copy(dst_hbm.at[pl.ds(base, CHUNK)], didx,
                                  sem_d).wait()
            cg_m = pltpu.async_copy(mesh_hbm.at[sidx], rows_m, sem_gm)
            cg_g = pltpu.async_copy(grid_hbm.at[didx], rows_g, sem_gg)
            cg_m.wait()
            cg_g.wait()
            cs_m = pltpu.async_copy(rows_m, gm_hbm.at[pl.ds(base, CHUNK)],
                                    sem_sm)
            cs_g = pltpu.async_copy(rows_g, gg_hbm.at[pl.ds(base, CHUNK)],
                                    sem_sg)
            cs_m.wait()
            cs_g.wait()

    idx_load(0, sidx0, didx0, sem_si0, sem_di0)

    def body(q, carry):
        i0 = 2 * q
        i1 = 2 * q + 1
        idx_load(i1, sidx1, didx1, sem_si1, sem_di1)
        process(i0, sidx0, didx0, sem_si0, sem_di0)
        idx_load(i1 + 1, sidx0, didx0, sem_si0, sem_di0)
        process(i1, sidx1, didx1, sem_si1, sem_di1)
        return carry

    lax.fori_loop(0, npairs, body, 0)


@functools.cache
def _make_gather():
    mesh = plsc.VectorSubcoreMesh(
        core_axis_name="c", subcore_axis_name="s",
        num_cores=NC, num_subcores=NS)
    return pl.kernel(
        _gather_body,
        out_type=[jax.ShapeDtypeStruct((E, D), jnp.float32),
                  jax.ShapeDtypeStruct((E, D), jnp.float32)],
        mesh=mesh,
        scratch_types=[pltpu.VMEM((CHUNK,), jnp.int32),
                       pltpu.VMEM((CHUNK,), jnp.int32),
                       pltpu.VMEM((CHUNK,), jnp.int32),
                       pltpu.VMEM((CHUNK,), jnp.int32),
                       pltpu.VMEM((CHUNK, D), jnp.float32),
                       pltpu.VMEM((CHUNK, D), jnp.float32),
                       pltpu.SemaphoreType.DMA,
                       pltpu.SemaphoreType.DMA,
                       pltpu.SemaphoreType.DMA,
                       pltpu.SemaphoreType.DMA,
                       pltpu.SemaphoreType.DMA,
                       pltpu.SemaphoreType.DMA,
                       pltpu.SemaphoreType.DMA,
                       pltpu.SemaphoreType.DMA],
    )


def _scatter_chunk(didx_v, rows_v, acc, lo):
    for v in range(CHUNK // 16):
        d16 = didx_v[pl.ds(v * 16, 16)]
        loc = d16 - lo
        m = (loc >= 0) & (loc < NHALF)
        loc = jnp.where(m, loc, 0)
        for j in range(FCOL):
            vals = rows_v[j, pl.ds(v * 16, 16)]
            plsc.addupdate_scatter(
                acc, [jnp.full((16,), j, jnp.int32), loc], vals, mask=m)


def _scatter_body(rows_hbm, dst_hbm, zeros_hbm, agg_hbm,
                  rows_v0, rows_v1, didx_v0, didx_v1, acc,
                  sem_r0, sem_r1, sem_i0, sem_i1):
    c = lax.axis_index("c")
    s = lax.axis_index("s")
    lo = c * NHALF
    col0 = s * FCOL

    pltpu.sync_copy(zeros_hbm, acc)

    def load(base, didx_v, rows_v, sem_i, sem_r):
        cp_i = pltpu.async_copy(dst_hbm.at[pl.ds(base, CHUNK)], didx_v, sem_i)
        cp_r = pltpu.async_copy(
            rows_hbm.at[pl.ds(col0, FCOL), pl.ds(base, CHUNK)], rows_v, sem_r)
        return cp_i, cp_r

    def wait(base, didx_v, rows_v, sem_i, sem_r):
        pltpu.make_async_copy(dst_hbm.at[pl.ds(base, CHUNK)], didx_v,
                              sem_i).wait()
        pltpu.make_async_copy(
            rows_hbm.at[pl.ds(col0, FCOL), pl.ds(base, CHUNK)], rows_v,
            sem_r).wait()

    load(0, didx_v0, rows_v0, sem_i0, sem_r0)

    def body(p, carry):
        base0 = (2 * p) * CHUNK
        base1 = (2 * p + 1) * CHUNK

        load(base1, didx_v1, rows_v1, sem_i1, sem_r1)
        wait(base0, didx_v0, rows_v0, sem_i0, sem_r0)
        _scatter_chunk(didx_v0, rows_v0, acc, lo)

        @pl.when(p + 1 < NPAIRS)
        def _():
            load(base0 + 2 * CHUNK, didx_v0, rows_v0, sem_i0, sem_r0)

        wait(base1, didx_v1, rows_v1, sem_i1, sem_r1)
        _scatter_chunk(didx_v1, rows_v1, acc, lo)
        return carry

    lax.fori_loop(0, NPAIRS, body, 0)

    pltpu.sync_copy(acc, agg_hbm.at[pl.ds(col0, FCOL), pl.ds(lo, NHALF)])


@functools.cache
def _make_scatter():
    mesh = plsc.VectorSubcoreMesh(
        core_axis_name="c", subcore_axis_name="s",
        num_cores=NC, num_subcores=NS)
    return pl.kernel(
        _scatter_body,
        out_type=[jax.ShapeDtypeStruct((D, NPAD), jnp.float32)],
        mesh=mesh,
        compiler_params=pltpu.CompilerParams(needs_layout_passes=False),
        scratch_types=[pltpu.VMEM((FCOL, CHUNK), jnp.float32),
                       pltpu.VMEM((FCOL, CHUNK), jnp.float32),
                       pltpu.VMEM((CHUNK,), jnp.int32),
                       pltpu.VMEM((CHUNK,), jnp.int32),
                       pltpu.VMEM((FCOL, NHALF), jnp.float32),
                       pltpu.SemaphoreType.DMA,
                       pltpu.SemaphoreType.DMA,
                       pltpu.SemaphoreType.DMA,
                       pltpu.SemaphoreType.DMA],
    )


BE = 1280
BN = 1024

_F32 = jnp.float32


def _edge_body(ef_ref, gm_ref, gg_ref, w1_ref, b1_ref, w2_ref, b2_ref,
               g_ref, bt_ref, out_ref):
    dn = (((0,), (1,)), ((), ()))
    x = lax.dot_general(w1_ref[0:D, :], ef_ref[...], dn,
                        preferred_element_type=_F32)
    x += lax.dot_general(w1_ref[D:2 * D, :], gm_ref[...], dn,
                         preferred_element_type=_F32)
    x += lax.dot_general(w1_ref[2 * D:3 * D, :], gg_ref[...], dn,
                         preferred_element_type=_F32)
    x += b1_ref[...]
    h = jax.nn.silu(x)
    y = lax.dot_general(w2_ref[...], h, (((0,), (0,)), ((), ())),
                        preferred_element_type=_F32) + b2_ref[...]
    mu = jnp.mean(y, axis=0, keepdims=True)
    var = jnp.mean((y - mu) ** 2, axis=0, keepdims=True)
    out_ref[...] = (y - mu) * lax.rsqrt(var + 1e-5) * g_ref[...] + bt_ref[...]


def _node_body(aggt_ref, gn_ref, w1_ref, b1_ref, w2_ref, b2_ref,
               g_ref, bt_ref, out_ref):
    x = lax.dot_general(aggt_ref[...], w1_ref[0:D, :], (((0,), (0,)), ((), ())),
                        preferred_element_type=_F32)
    x += jnp.dot(gn_ref[...], w1_ref[D:2 * D, :], preferred_element_type=_F32)
    x += b1_ref[...]
    h = jax.nn.silu(x)
    y = jnp.dot(h, w2_ref[...], preferred_element_type=_F32) + b2_ref[...]
    mu = jnp.mean(y, axis=-1, keepdims=True)
    var = jnp.mean((y - mu) ** 2, axis=-1, keepdims=True)
    out_ref[...] = ((y - mu) * lax.rsqrt(var + 1e-5) * g_ref[...] + bt_ref[...]
                    + gn_ref[...])


def _row_spec(r, c_):
    return pl.BlockSpec((r, c_), lambda i: (i, 0))


def _col_spec(r, c_):
    return pl.BlockSpec((r, c_), lambda i: (0, i))


def _full_spec(r, c_):
    return pl.BlockSpec((r, c_), lambda i: (0, 0))


_edge_mlp = pl.pallas_call(
    _edge_body,
    grid=(E // BE,),
    in_specs=[_row_spec(BE, D), _row_spec(BE, D), _row_spec(BE, D),
              _full_spec(3 * D, H), _full_spec(H, 1),
              _full_spec(H, D), _full_spec(D, 1),
              _full_spec(D, 1), _full_spec(D, 1)],
    out_specs=_col_spec(D, BE),
    out_shape=jax.ShapeDtypeStruct((D, E), jnp.float32),
)

_node_mlp = pl.pallas_call(
    _node_body,
    grid=(NPAD // BN,),
    in_specs=[_col_spec(D, BN), _row_spec(BN, D),
              _full_spec(2 * D, H), _full_spec(1, H),
              _full_spec(H, D), _full_spec(1, D),
              _full_spec(1, D), _full_spec(1, D)],
    out_specs=_row_spec(BN, D),
    out_shape=jax.ShapeDtypeStruct((NPAD, D), jnp.float32),
)


def kernel(m2g_efeat, grid_nfeat, mesh_nfeat, edge_index,
           eW1, eb1, eW2, eb2, eg, ebt,
           nW1, nb1, nW2, nb2, ng, nbt):
    src = edge_index[0]
    dst = edge_index[1]

    gm, gg = _make_gather()(mesh_nfeat, grid_nfeat, src, dst)

    e_out_t = _edge_mlp(m2g_efeat, gm, gg, eW1,
                        eb1.reshape(H, 1), eW2, eb2.reshape(D, 1),
                        eg.reshape(D, 1), ebt.reshape(D, 1))

    zeros = jnp.zeros((FCOL, NHALF), dtype=jnp.float32)
    (agg_t,) = _make_scatter()(e_out_t, dst, zeros)

    gn_pad = jnp.pad(grid_nfeat, ((0, NPAD - N_GRID), (0, 0)))
    out = _node_mlp(agg_t, gn_pad, nW1,
                    nb1.reshape(1, H), nW2, nb2.reshape(1, D),
                    ng.reshape(1, D), nbt.reshape(1, D))
    return out[:N_GRID]


# speedup vs baseline: 2.3125x; 1.1646x over previous
import functools

import jax
import jax.numpy as jnp
from jax import lax
from jax.experimental import pallas as pl
from jax.experimental.pallas import tpu as pltpu
from jax.experimental.pallas import tpu_sc as plsc

N_MESH = 10000
N_GRID = 10000
E = 160000
D = 256
H = 512

NC = 2
NS = 16
NW = NC * NS
CHUNK = 128
NCHUNKS = E // CHUNK
NPAIRS = NCHUNKS // 2

NPAD = 10240
NHALF = NPAD // NC
FCOL = D // NS


def _gather_body(mesh_hbm, grid_hbm, src_hbm, dst_hbm, gm_hbm, gg_hbm,
                 sidx0, didx0, sidx1, didx1, rows_m, rows_g,
                 sem_si0, sem_di0, sem_si1, sem_di1,
                 sem_gm, sem_gg, sem_sm, sem_sg):
    c = lax.axis_index("c")
    s = lax.axis_index("s")
    wid = s * NC + c
    niters = (NCHUNKS + NW - 1) // NW
    npairs = (niters + 1) // 2

    def idx_load(i, sidx, didx, sem_s, sem_d):
        chunk = wid + i * NW

        @pl.when(chunk < NCHUNKS)
        def _():
            base = chunk * CHUNK
            pltpu.async_copy(src_hbm.at[pl.ds(base, CHUNK)], sidx, sem_s)
            pltpu.async_copy(dst_hbm.at[pl.ds(base, CHUNK)], didx, sem_d)

    def process(i, sidx, didx, sem_s, sem_d):
        chunk = wid + i * NW

        @pl.when(chunk < NCHUNKS)
        def _():
            base = chunk * CHUNK
            pltpu.make_async_copy(src_hbm.at[pl.ds(base, CHUNK)], sidx,
                                  sem_s).wait()
            pltpu.make_async_copy(dst_hbm.at[pl.ds(base, CHUNK)], didx,
                                  sem_d).wait()
            cg_m = pltpu.async_copy(mesh_hbm.at[sidx], rows_m, sem_gm)
            cg_g = pltpu.async_copy(grid_hbm.at[didx], rows_g, sem_gg)
            cg_m.wait()
            cg_g.wait()
            cs_m = pltpu.async_copy(rows_m, gm_hbm.at[pl.ds(base, CHUNK)],
                                    sem_sm)
            cs_g = pltpu.async_copy(rows_g, gg_hbm.at[pl.ds(base, CHUNK)],
                                    sem_sg)
            cs_m.wait()
            cs_g.wait()

    idx_load(0, sidx0, didx0, sem_si0, sem_di0)

    def body(q, carry):
        i0 = 2 * q
        i1 = 2 * q + 1
        idx_load(i1, sidx1, didx1, sem_si1, sem_di1)
        process(i0, sidx0, didx0, sem_si0, sem_di0)
        idx_load(i1 + 1, sidx0, didx0, sem_si0, sem_di0)
        process(i1, sidx1, didx1, sem_si1, sem_di1)
        return carry

    lax.fori_loop(0, npairs, body, 0)


@functools.cache
def _make_gather():
    mesh = plsc.VectorSubcoreMesh(
        core_axis_name="c", subcore_axis_name="s",
        num_cores=NC, num_subcores=NS)
    return pl.kernel(
        _gather_body,
        out_type=[jax.ShapeDtypeStruct((E, D), jnp.float32),
                  jax.ShapeDtypeStruct((E, D), jnp.float32)],
        mesh=mesh,
        scratch_types=[pltpu.VMEM((CHUNK,), jnp.int32),
                       pltpu.VMEM((CHUNK,), jnp.int32),
                       pltpu.VMEM((CHUNK,), jnp.int32),
                       pltpu.VMEM((CHUNK,), jnp.int32),
                       pltpu.VMEM((CHUNK, D), jnp.float32),
                       pltpu.VMEM((CHUNK, D), jnp.float32),
                       pltpu.SemaphoreType.DMA,
                       pltpu.SemaphoreType.DMA,
                       pltpu.SemaphoreType.DMA,
                       pltpu.SemaphoreType.DMA,
                       pltpu.SemaphoreType.DMA,
                       pltpu.SemaphoreType.DMA,
                       pltpu.SemaphoreType.DMA,
                       pltpu.SemaphoreType.DMA],
    )


FROW = D // NW


def _scatter_chunk(didx_v, rows_v, acc):
    for v in range(CHUNK // 16):
        d16 = didx_v[pl.ds(v * 16, 16)]
        for j in range(FROW):
            vals = rows_v[j, pl.ds(v * 16, 16)]
            plsc.addupdate_scatter(
                acc, [jnp.full((16,), j, jnp.int32), d16], vals)


def _scatter_body(rows_hbm, dst_hbm, zeros_hbm, agg_hbm,
                  rows_v0, rows_v1, didx_v0, didx_v1, acc,
                  sem_r0, sem_r1, sem_i0, sem_i1):
    c = lax.axis_index("c")
    s = lax.axis_index("s")
    fid = s * NC + c
    col0 = fid * FROW

    pltpu.sync_copy(zeros_hbm, acc)

    def load(base, didx_v, rows_v, sem_i, sem_r):
        pltpu.async_copy(dst_hbm.at[pl.ds(base, CHUNK)], didx_v, sem_i)
        pltpu.async_copy(
            rows_hbm.at[pl.ds(col0, FROW), pl.ds(base, CHUNK)], rows_v, sem_r)

    def wait(base, didx_v, rows_v, sem_i, sem_r):
        pltpu.make_async_copy(dst_hbm.at[pl.ds(base, CHUNK)], didx_v,
                              sem_i).wait()
        pltpu.make_async_copy(
            rows_hbm.at[pl.ds(col0, FROW), pl.ds(base, CHUNK)], rows_v,
            sem_r).wait()

    load(0, didx_v0, rows_v0, sem_i0, sem_r0)

    def body(p, carry):
        base0 = (2 * p) * CHUNK
        base1 = (2 * p + 1) * CHUNK

        load(base1, didx_v1, rows_v1, sem_i1, sem_r1)
        wait(base0, didx_v0, rows_v0, sem_i0, sem_r0)
        _scatter_chunk(didx_v0, rows_v0, acc)

        @pl.when(p + 1 < NPAIRS)
        def _():
            load(base0 + 2 * CHUNK, didx_v0, rows_v0, sem_i0, sem_r0)

        wait(base1, didx_v1, rows_v1, sem_i1, sem_r1)
        _scatter_chunk(didx_v1, rows_v1, acc)
        return carry

    lax.fori_loop(0, NPAIRS, body, 0)

    pltpu.sync_copy(acc, agg_hbm.at[pl.ds(col0, FROW)])


@functools.cache
def _make_scatter():
    mesh = plsc.VectorSubcoreMesh(
        core_axis_name="c", subcore_axis_name="s",
        num_cores=NC, num_subcores=NS)
    return pl.kernel(
        _scatter_body,
        out_type=[jax.ShapeDtypeStruct((D, NPAD), jnp.float32)],
        mesh=mesh,
        compiler_params=pltpu.CompilerParams(needs_layout_passes=False),
        scratch_types=[pltpu.VMEM((FROW, CHUNK), jnp.float32),
                       pltpu.VMEM((FROW, CHUNK), jnp.float32),
                       pltpu.VMEM((CHUNK,), jnp.int32),
                       pltpu.VMEM((CHUNK,), jnp.int32),
                       pltpu.VMEM((FROW, NPAD), jnp.float32),
                       pltpu.SemaphoreType.DMA,
                       pltpu.SemaphoreType.DMA,
                       pltpu.SemaphoreType.DMA,
                       pltpu.SemaphoreType.DMA],
    )


BE = 1280
BN = 1024

_F32 = jnp.float32


def _edge_body(ef_ref, gm_ref, gg_ref, w1_ref, b1_ref, w2_ref, b2_ref,
               g_ref, bt_ref, out_ref):
    dn = (((0,), (1,)), ((), ()))
    x = lax.dot_general(w1_ref[0:D, :], ef_ref[...], dn,
                        preferred_element_type=_F32)
    x += lax.dot_general(w1_ref[D:2 * D, :], gm_ref[...], dn,
                         preferred_element_type=_F32)
    x += lax.dot_general(w1_ref[2 * D:3 * D, :], gg_ref[...], dn,
                         preferred_element_type=_F32)
    x += b1_ref[...]
    h = jax.nn.silu(x)
    y = lax.dot_general(w2_ref[...], h, (((0,), (0,)), ((), ())),
                        preferred_element_type=_F32) + b2_ref[...]
    mu = jnp.mean(y, axis=0, keepdims=True)
    var = jnp.mean((y - mu) ** 2, axis=0, keepdims=True)
    out_ref[...] = (y - mu) * lax.rsqrt(var + 1e-5) * g_ref[...] + bt_ref[...]


def _node_body(aggt_ref, gn_ref, w1_ref, b1_ref, w2_ref, b2_ref,
               g_ref, bt_ref, out_ref):
    x = lax.dot_general(aggt_ref[...], w1_ref[0:D, :], (((0,), (0,)), ((), ())),
                        preferred_element_type=_F32)
    x += jnp.dot(gn_ref[...], w1_ref[D:2 * D, :], preferred_element_type=_F32)
    x += b1_ref[...]
    h = jax.nn.silu(x)
    y = jnp.dot(h, w2_ref[...], preferred_element_type=_F32) + b2_ref[...]
    mu = jnp.mean(y, axis=-1, keepdims=True)
    var = jnp.mean((y - mu) ** 2, axis=-1, keepdims=True)
    out_ref[...] = ((y - mu) * lax.rsqrt(var + 1e-5) * g_ref[...] + bt_ref[...]
                    + gn_ref[...])


def _row_spec(r, c_):
    return pl.BlockSpec((r, c_), lambda i: (i, 0))


def _col_spec(r, c_):
    return pl.BlockSpec((r, c_), lambda i: (0, i))


def _full_spec(r, c_):
    return pl.BlockSpec((r, c_), lambda i: (0, 0))


_edge_mlp = pl.pallas_call(
    _edge_body,
    grid=(E // BE,),
    in_specs=[_row_spec(BE, D), _row_spec(BE, D), _row_spec(BE, D),
              _full_spec(3 * D, H), _full_spec(H, 1),
              _full_spec(H, D), _full_spec(D, 1),
              _full_spec(D, 1), _full_spec(D, 1)],
    out_specs=_col_spec(D, BE),
    out_shape=jax.ShapeDtypeStruct((D, E), jnp.float32),
)

_node_mlp = pl.pallas_call(
    _node_body,
    grid=(NPAD // BN,),
    in_specs=[_col_spec(D, BN), _row_spec(BN, D),
              _full_spec(2 * D, H), _full_spec(1, H),
              _full_spec(H, D), _full_spec(1, D),
              _full_spec(1, D), _full_spec(1, D)],
    out_specs=_row_spec(BN, D),
    out_shape=jax.ShapeDtypeStruct((NPAD, D), jnp.float32),
)


def kernel(m2g_efeat, grid_nfeat, mesh_nfeat, edge_index,
           eW1, eb1, eW2, eb2, eg, ebt,
           nW1, nb1, nW2, nb2, ng, nbt):
    src = edge_index[0]
    dst = edge_index[1]

    gm, gg = _make_gather()(mesh_nfeat, grid_nfeat, src, dst)

    e_out_t = _edge_mlp(m2g_efeat, gm, gg, eW1,
                        eb1.reshape(H, 1), eW2, eb2.reshape(D, 1),
                        eg.reshape(D, 1), ebt.reshape(D, 1))

    zeros = jnp.zeros((FROW, NPAD), dtype=jnp.float32)
    (agg_t,) = _make_scatter()(e_out_t, dst, zeros)

    gn_pad = jnp.pad(grid_nfeat, ((0, NPAD - N_GRID), (0, 0)))
    out = _node_mlp(agg_t, gn_pad, nW1,
                    nb1.reshape(1, H), nW2, nb2.reshape(1, D),
                    ng.reshape(1, D), nbt.reshape(1, D))
    return out[:N_GRID]


# speedup vs baseline: 2.6589x; 1.1498x over previous
import functools

import jax
import jax.numpy as jnp
from jax import lax
from jax.experimental import pallas as pl
from jax.experimental.pallas import tpu as pltpu
from jax.experimental.pallas import tpu_sc as plsc

N_MESH = 10000
N_GRID = 10000
E = 160000
D = 256
H = 512

NC = 2
NS = 16
NW = NC * NS
CHUNK = 128
NCHUNKS = E // CHUNK
NPAIRS = NCHUNKS // 2

NPAD = 10240
NHALF = NPAD // NC
FCOL = D // NS


def _gather_body(mesh_hbm, grid_hbm, src_hbm, dst_hbm, gm_hbm, gg_hbm,
                 sidx0, didx0, sidx1, didx1, rows_m, rows_g,
                 sem_si0, sem_di0, sem_si1, sem_di1,
                 sem_gm, sem_gg, sem_sm, sem_sg):
    c = lax.axis_index("c")
    s = lax.axis_index("s")
    wid = s * NC + c
    niters = (NCHUNKS + NW - 1) // NW
    npairs = (niters + 1) // 2

    def idx_load(i, sidx, didx, sem_s, sem_d):
        chunk = wid + i * NW

        @pl.when(chunk < NCHUNKS)
        def _():
            base = chunk * CHUNK
            pltpu.async_copy(src_hbm.at[pl.ds(base, CHUNK)], sidx, sem_s)
            pltpu.async_copy(dst_hbm.at[pl.ds(base, CHUNK)], didx, sem_d)

    def process(i, sidx, didx, sem_s, sem_d):
        chunk = wid + i * NW

        @pl.when(chunk < NCHUNKS)
        def _():
            base = chunk * CHUNK
            pltpu.make_async_copy(src_hbm.at[pl.ds(base, CHUNK)], sidx,
                                  sem_s).wait()
            pltpu.make_async_copy(dst_hbm.at[pl.ds(base, CHUNK)], didx,
                                  sem_d).wait()
            cg_m = pltpu.async_copy(mesh_hbm.at[sidx], rows_m, sem_gm)
            cg_g = pltpu.async_copy(grid_hbm.at[didx], rows_g, sem_gg)
            cg_m.wait()
            cg_g.wait()
            cs_m = pltpu.async_copy(rows_m, gm_hbm.at[pl.ds(base, CHUNK)],
                                    sem_sm)
            cs_g = pltpu.async_copy(rows_g, gg_hbm.at[pl.ds(base, CHUNK)],
                                    sem_sg)
            cs_m.wait()
            cs_g.wait()

    idx_load(0, sidx0, didx0, sem_si0, sem_di0)

    def body(q, carry):
        i0 = 2 * q
        i1 = 2 * q + 1
        idx_load(i1, sidx1, didx1, sem_si1, sem_di1)
        process(i0, sidx0, didx0, sem_si0, sem_di0)
        idx_load(i1 + 1, sidx0, didx0, sem_si0, sem_di0)
        process(i1, sidx1, didx1, sem_si1, sem_di1)
        return carry

    lax.fori_loop(0, npairs, body, 0)


@functools.cache
def _make_gather():
    mesh = plsc.VectorSubcoreMesh(
        core_axis_name="c", subcore_axis_name="s",
        num_cores=NC, num_subcores=NS)
    return pl.kernel(
        _gather_body,
        out_type=[jax.ShapeDtypeStruct((E, D), jnp.float32),
                  jax.ShapeDtypeStruct((E, D), jnp.float32)],
        mesh=mesh,
        scratch_types=[pltpu.VMEM((CHUNK,), jnp.int32),
                       pltpu.VMEM((CHUNK,), jnp.int32),
                       pltpu.VMEM((CHUNK,), jnp.int32),
                       pltpu.VMEM((CHUNK,), jnp.int32),
                       pltpu.VMEM((CHUNK, D), jnp.float32),
                       pltpu.VMEM((CHUNK, D), jnp.float32),
                       pltpu.SemaphoreType.DMA,
                       pltpu.SemaphoreType.DMA,
                       pltpu.SemaphoreType.DMA,
                       pltpu.SemaphoreType.DMA,
                       pltpu.SemaphoreType.DMA,
                       pltpu.SemaphoreType.DMA,
                       pltpu.SemaphoreType.DMA,
                       pltpu.SemaphoreType.DMA],
    )


FROW = D // NW
SCHUNK = 256
SNCHUNKS = E // SCHUNK
SNPAIRS = SNCHUNKS // 2


def _scatter_chunk(didx_v, rows_v, acc):
    for v in range(SCHUNK // 16):
        d16 = didx_v[pl.ds(v * 16, 16)]
        for j in range(FROW):
            vals = rows_v[j, pl.ds(v * 16, 16)]
            plsc.addupdate_scatter(
                acc, [jnp.full((16,), j, jnp.int32), d16], vals)


def _scatter_body(rows_hbm, dst_hbm, zeros_hbm, agg_hbm,
                  rows_v0, rows_v1, didx_v0, didx_v1, acc,
                  sem_r0, sem_r1, sem_i0, sem_i1):
    c = lax.axis_index("c")
    s = lax.axis_index("s")
    fid = s * NC + c
    col0 = fid * FROW

    pltpu.sync_copy(zeros_hbm, acc)

    def load(base, didx_v, rows_v, sem_i, sem_r):
        pltpu.async_copy(dst_hbm.at[pl.ds(base, SCHUNK)], didx_v, sem_i)
        pltpu.async_copy(
            rows_hbm.at[pl.ds(col0, FROW), pl.ds(base, SCHUNK)], rows_v,
            sem_r)

    def wait(base, didx_v, rows_v, sem_i, sem_r):
        pltpu.make_async_copy(dst_hbm.at[pl.ds(base, SCHUNK)], didx_v,
                              sem_i).wait()
        pltpu.make_async_copy(
            rows_hbm.at[pl.ds(col0, FROW), pl.ds(base, SCHUNK)], rows_v,
            sem_r).wait()

    load(0, didx_v0, rows_v0, sem_i0, sem_r0)

    def body(p, carry):
        base0 = (2 * p) * SCHUNK
        base1 = (2 * p + 1) * SCHUNK

        load(base1, didx_v1, rows_v1, sem_i1, sem_r1)
        wait(base0, didx_v0, rows_v0, sem_i0, sem_r0)
        _scatter_chunk(didx_v0, rows_v0, acc)

        load(base0 + 2 * SCHUNK, didx_v0, rows_v0, sem_i0, sem_r0)

        wait(base1, didx_v1, rows_v1, sem_i1, sem_r1)
        _scatter_chunk(didx_v1, rows_v1, acc)
        return carry

    lax.fori_loop(0, SNPAIRS, body, 0)

    base_l = (SNCHUNKS - 1) * SCHUNK
    wait(base_l, didx_v0, rows_v0, sem_i0, sem_r0)
    _scatter_chunk(didx_v0, rows_v0, acc)

    pltpu.sync_copy(acc, agg_hbm.at[pl.ds(col0, FROW)])


@functools.cache
def _make_scatter():
    mesh = plsc.VectorSubcoreMesh(
        core_axis_name="c", subcore_axis_name="s",
        num_cores=NC, num_subcores=NS)
    return pl.kernel(
        _scatter_body,
        out_type=[jax.ShapeDtypeStruct((D, NPAD), jnp.float32)],
        mesh=mesh,
        compiler_params=pltpu.CompilerParams(needs_layout_passes=False),
        scratch_types=[pltpu.VMEM((FROW, SCHUNK), jnp.float32),
                       pltpu.VMEM((FROW, SCHUNK), jnp.float32),
                       pltpu.VMEM((SCHUNK,), jnp.int32),
                       pltpu.VMEM((SCHUNK,), jnp.int32),
                       pltpu.VMEM((FROW, NPAD), jnp.float32),
                       pltpu.SemaphoreType.DMA,
                       pltpu.SemaphoreType.DMA,
                       pltpu.SemaphoreType.DMA,
                       pltpu.SemaphoreType.DMA],
    )


BE = 1280
BN = 1024

_F32 = jnp.float32


def _edge_body(ef_ref, gm_ref, gg_ref, w1_ref, b1_ref, w2_ref, b2_ref,
               g_ref, bt_ref, out_ref):
    dn = (((0,), (1,)), ((), ()))
    x = lax.dot_general(w1_ref[0:D, :], ef_ref[...], dn,
                        preferred_element_type=_F32)
    x += lax.dot_general(w1_ref[D:2 * D, :], gm_ref[...], dn,
                         preferred_element_type=_F32)
    x += lax.dot_general(w1_ref[2 * D:3 * D, :], gg_ref[...], dn,
                         preferred_element_type=_F32)
    x += b1_ref[...]
    h = jax.nn.silu(x)
    y = lax.dot_general(w2_ref[...], h, (((0,), (0,)), ((), ())),
                        preferred_element_type=_F32) + b2_ref[...]
    mu = jnp.mean(y, axis=0, keepdims=True)
    var = jnp.mean((y - mu) ** 2, axis=0, keepdims=True)
    out_ref[...] = (y - mu) * lax.rsqrt(var + 1e-5) * g_ref[...] + bt_ref[...]


def _node_body(aggt_ref, gn_ref, w1_ref, b1_ref, w2_ref, b2_ref,
               g_ref, bt_ref, out_ref):
    x = lax.dot_general(aggt_ref[...], w1_ref[0:D, :], (((0,), (0,)), ((), ())),
                        preferred_element_type=_F32)
    x += jnp.dot(gn_ref[...], w1_ref[D:2 * D, :], preferred_element_type=_F32)
    x += b1_ref[...]
    h = jax.nn.silu(x)
    y = jnp.dot(h, w2_ref[...], preferred_element_type=_F32) + b2_ref[...]
    mu = jnp.mean(y, axis=-1, keepdims=True)
    var = jnp.mean((y - mu) ** 2, axis=-1, keepdims=True)
    out_ref[...] = ((y - mu) * lax.rsqrt(var + 1e-5) * g_ref[...] + bt_ref[...]
                    + gn_ref[...])


def _row_spec(r, c_):
    return pl.BlockSpec((r, c_), lambda i: (i, 0))


def _col_spec(r, c_):
    return pl.BlockSpec((r, c_), lambda i: (0, i))


def _full_spec(r, c_):
    return pl.BlockSpec((r, c_), lambda i: (0, 0))


_edge_mlp = pl.pallas_call(
    _edge_body,
    grid=(E // BE,),
    in_specs=[_row_spec(BE, D), _row_spec(BE, D), _row_spec(BE, D),
              _full_spec(3 * D, H), _full_spec(H, 1),
              _full_spec(H, D), _full_spec(D, 1),
              _full_spec(D, 1), _full_spec(D, 1)],
    out_specs=_col_spec(D, BE),
    out_shape=jax.ShapeDtypeStruct((D, E), jnp.float32),
)

_node_mlp = pl.pallas_call(
    _node_body,
    grid=(NPAD // BN,),
    in_specs=[_col_spec(D, BN), _row_spec(BN, D),
              _full_spec(2 * D, H), _full_spec(1, H),
              _full_spec(H, D), _full_spec(1, D),
              _full_spec(1, D), _full_spec(1, D)],
    out_specs=_row_spec(BN, D),
    out_shape=jax.ShapeDtypeStruct((NPAD, D), jnp.float32),
)


def kernel(m2g_efeat, grid_nfeat, mesh_nfeat, edge_index,
           eW1, eb1, eW2, eb2, eg, ebt,
           nW1, nb1, nW2, nb2, ng, nbt):
    src = edge_index[0]
    dst = edge_index[1]

    gm, gg = _make_gather()(mesh_nfeat, grid_nfeat, src, dst)

    e_out_t = _edge_mlp(m2g_efeat, gm, gg, eW1,
                        eb1.reshape(H, 1), eW2, eb2.reshape(D, 1),
                        eg.reshape(D, 1), ebt.reshape(D, 1))

    zeros = jnp.zeros((FROW, NPAD), dtype=jnp.float32)
    (agg_t,) = _make_scatter()(e_out_t, dst, zeros)

    gn_pad = jnp.pad(grid_nfeat, ((0, NPAD - N_GRID), (0, 0)))
    out = _node_mlp(agg_t, gn_pad, nW1,
                    nb1.reshape(1, H), nW2, nb2.reshape(1, D),
                    ng.reshape(1, D), nbt.reshape(1, D))
    return out[:N_GRID]
